# baseline (speedup 1.0000x reference)
import functools

import jax
import jax.numpy as jnp
from jax import lax
from jax.experimental import pallas as pl
from jax.experimental.pallas import tpu as pltpu
from jax.experimental.pallas import tpu_sc as plsc

D = 32
B = 16384 * 50

NC = 2
NS = 16
NW = NC * NS
B_PER_W = B // NW

G = 128
HALF = 4 * G
N_HALVES = B_PER_W // HALF
N_BLOCKS = N_HALVES // 2

_mesh = plsc.VectorSubcoreMesh(core_axis_name="c", subcore_axis_name="s")


@functools.partial(
    pl.kernel,
    mesh=_mesh,
    out_type=jax.ShapeDtypeStruct((B // 4, 4 * D), jnp.float32),
    scratch_types=[
        pltpu.VMEM((8, G), jnp.int32),
        pltpu.VMEM((8, G), jnp.int32),
        pltpu.VMEM((HALF, D), jnp.float32),
        pltpu.VMEM((HALF, D), jnp.float32),
        pltpu.VMEM((HALF // 4, 4 * D), jnp.float32),
        pltpu.VMEM((HALF // 4, 4 * D), jnp.float32),
        pltpu.SemaphoreType.DMA,
        pltpu.SemaphoreType.DMA,
        pltpu.SemaphoreType.DMA,
        pltpu.SemaphoreType.DMA,
        pltpu.SemaphoreType.DMA,
        pltpu.SemaphoreType.DMA,
    ],
    compiler_params=pltpu.CompilerParams(use_tc_tiling_on_sc=False),
)
def _gather_kernel(idx_hbm, table_hbm, out_hbm,
                   idxA, idxB, rows0, rows1, pack0, pack1,
                   isA, isB, gs0, gs1, os0, os1):
    wid = lax.axis_index("s") * NC + lax.axis_index("c")
    idx_row0 = wid * (B_PER_W // G)
    out_row0 = wid * (B_PER_W // 4)

    idx_v = (idxA, idxB)
    rows_v = (rows0, rows1)
    pack_v = (pack0, pack1)
    isem = (isA, isB)
    gsem = (gs0, gs1)
    osem = (os0, os1)

    def idx_load(tb, p):
        pltpu.make_async_copy(
            idx_hbm.at[pl.ds(idx_row0 + tb * 8, 8)], idx_v[p], isem[p]).start()

    def idx_wait(p):
        pltpu.make_async_copy(idx_hbm.at[pl.ds(0, 8)], idx_v[p], isem[p]).wait()

    def fire(rb, p, jbase):
        for j in range(4):
            pltpu.make_async_copy(
                table_hbm.at[idx_v[p].at[jbase + j]],
                rows_v[rb].at[pl.ds(j * G, G)], gsem[rb]).start()

    def drain(rb):
        pltpu.make_async_copy(
            table_hbm.at[pl.ds(0, HALF)], rows_v[rb], gsem[rb]).wait()

    def repack(rb):
        def body(r, _):
            for u in range(8):
                pack_v[rb][r, pl.ds(u * 16, 16)] = (
                    rows_v[rb][r * 4 + u // 2, pl.ds((u % 2) * 16, 16)])
            return 0
        lax.fori_loop(0, HALF // 4, body, 0)

    def out_start(h, rb):
        pltpu.make_async_copy(
            pack_v[rb],
            out_hbm.at[pl.ds(out_row0 + h * (HALF // 4), HALF // 4)],
            osem[rb]).start()

    def out_wait(rb):
        pltpu.make_async_copy(
            pack_v[rb], out_hbm.at[pl.ds(0, HALF // 4)], osem[rb]).wait()

    def finish_prev(h, rb_prev):
        drain(rb_prev)
        repack(rb_prev)
        out_start(h - 1, rb_prev)

    idx_load(0, 0)
    idx_load(1, 1)
    idx_wait(0)
    fire(0, 0, 0)

    def quad(T, _):
        h = 4 * T + 1

        @pl.when(T > 0)
        def _():
            out_wait(1)
        fire(1, 0, 4)
        finish_prev(h, 0)

        h = 4 * T + 2
        idx_wait(1)
        out_wait(0)
        fire(0, 1, 0)
        finish_prev(h, 1)
        idx_load(2 * T + 2, 0)

        h = 4 * T + 3
        out_wait(1)
        fire(1, 1, 4)
        finish_prev(h, 0)

        h = 4 * T + 4
        idx_wait(0)
        out_wait(0)
        fire(0, 0, 0)
        finish_prev(h, 1)

        @pl.when(T < N_BLOCKS // 2 - 1)
        def _():
            idx_load(2 * T + 3, 1)

        return 0

    lax.fori_loop(0, (N_HALVES - 2) // 4, quad, 0)

    out_wait(1)
    fire(1, 0, 4)
    drain(0)
    repack(0)
    out_start(48, 0)
    drain(1)
    repack(1)
    out_start(49, 1)
    out_wait(0)
    out_wait(1)


def kernel(camera_ids, weight):
    ids = camera_ids.reshape(-1).astype(jnp.int32)
    idx2d = ids.reshape(B // G, G)
    return _gather_kernel(idx2d, weight).reshape(B, D)
